# Initial kernel scaffold; baseline (speedup 1.0000x reference)
import jax
import jax.numpy as jnp
from jax.experimental import pallas as pl


def kernel(scores, logits, beam_width):
    raise NotImplementedError("write your pallas kernel here")


# speedup vs baseline: 49.1813x; 49.1813x over previous
import functools

import jax
import jax.numpy as jnp
from jax.experimental import pallas as pl


def _beam_step_kernel(scores_ref, logits_ref, out_s_ref, out_r_ref, out_t_ref):
    x = logits_ref[0]
    W, V = x.shape
    neg = jnp.float32(-jnp.inf)

    m = jnp.max(x, axis=-1, keepdims=True)
    s = jnp.sum(jnp.exp(x - m), axis=-1, keepdims=True)
    lse = m + jnp.log(s)

    col = jax.lax.broadcasted_iota(jnp.int32, (W, V), 1)
    work = x
    vals, toks = [], []
    for _ in range(W):
        cm = jnp.max(work, axis=-1, keepdims=True)
        ci = jnp.min(jnp.where(work == cm, col, V), axis=-1, keepdims=True)
        vals.append(cm)
        toks.append(ci)
        work = jnp.where(col == ci, neg, work)
    cand_v = jnp.concatenate(vals, axis=1)
    cand_t = jnp.concatenate(toks, axis=1)

    sc = scores_ref[0]
    ext = sc + cand_v - lse
    r_io = jax.lax.broadcasted_iota(jnp.int32, (W, W), 0)
    c_io = jax.lax.broadcasted_iota(jnp.int32, (W, W), 1)
    f_io = r_io * W + c_io
    out_v, out_r, out_t = [], [], []
    fw = ext
    for _ in range(W):
        fm = jnp.max(jnp.max(fw, axis=1, keepdims=True), axis=0, keepdims=True)
        fi = jnp.min(
            jnp.min(jnp.where(fw == fm, f_io, W * W), axis=1, keepdims=True),
            axis=0,
            keepdims=True,
        )
        fw = jnp.where(f_io == fi, neg, fw)
        onehot = f_io == fi
        tok = jnp.sum(
            jnp.sum(jnp.where(onehot, cand_t, 0), axis=1, keepdims=True),
            axis=0,
            keepdims=True,
        )
        out_v.append(fm)
        out_r.append(fi // W)
        out_t.append(tok)
    out_s_ref[0] = jnp.concatenate(out_v, axis=1)
    out_r_ref[0] = jnp.concatenate(out_r, axis=1)
    out_t_ref[0] = jnp.concatenate(out_t, axis=1)


def kernel(scores, logits, beam_width):
    del beam_width
    B, W, V = logits.shape
    scores3 = scores.reshape(B, W, 1)
    o_shape = jax.ShapeDtypeStruct((B, 1, W), jnp.float32)
    i_shape = jax.ShapeDtypeStruct((B, 1, W), jnp.int32)
    o_spec = pl.BlockSpec((1, 1, W), lambda b: (b, 0, 0))
    ps, pr, pt = pl.pallas_call(
        _beam_step_kernel,
        grid=(B,),
        in_specs=[
            pl.BlockSpec((1, W, 1), lambda b: (b, 0, 0)),
            pl.BlockSpec((1, W, V), lambda b: (b, 0, 0)),
        ],
        out_specs=(o_spec, o_spec, o_spec),
        out_shape=(o_shape, i_shape, i_shape),
    )(scores3, logits)
    return ps.reshape(B, W), pr.reshape(B, W), pt.reshape(B, W)


# speedup vs baseline: 77.2394x; 1.5705x over previous
import functools

import jax
import jax.numpy as jnp
from jax import lax
from jax.experimental import pallas as pl
from jax.experimental.pallas import tpu as pltpu
from jax.experimental.pallas import tpu_sc as plsc

_CH = 128
_L = 16


def _dense_kernel(logits_ref, lse_ref, cmax_ref, *, n_chunks):
    x = logits_ref[0]
    W, V = x.shape
    neg = jnp.float32(-jnp.inf)
    pad = n_chunks * _CH - V
    padded = jnp.concatenate([x, jnp.full((W, pad), neg, jnp.float32)], axis=1)
    cmax = jnp.max(padded.reshape(W, n_chunks, _CH), axis=-1)
    cmax_ref[0] = cmax
    m = jnp.max(cmax, axis=-1, keepdims=True)
    s = jnp.sum(jnp.exp(x - m), axis=-1, keepdims=True)
    lse_ref[0] = m + jnp.log(s)


def _lane_iota():
    return lax.iota(jnp.int32, _L)


def _first_argmax_scan(ref, n_vecs, idx_base_fn):
    def body(i, carry):
        runv, runi = carry
        v = ref[pl.ds(i * _L, _L)]
        idx = idx_base_fn(i) + _lane_iota()
        better = (v > runv) | ((v == runv) & (idx < runi))
        return jnp.where(better, v, runv), jnp.where(better, idx, runi)

    init = (jnp.full((_L,), -jnp.inf, jnp.float32), jnp.full((_L,), jnp.int32(2**30)))
    runv, runi = lax.fori_loop(0, n_vecs, body, init)
    mx = jnp.max(runv)
    mi = jnp.min(jnp.where(runv == mx, runi, jnp.int32(2**30)))
    return mx, mi


def _mask_out(ref, pos):
    lane0 = _lane_iota() == 0
    plsc.store_scatter(
        ref,
        [jnp.full((_L,), pos, jnp.int32)],
        jnp.full((_L,), -jnp.inf, jnp.float32),
        mask=lane0,
    )


def _scatter1(ref, pos, val):
    lane0 = _lane_iota() == 0
    plsc.store_scatter(ref, [jnp.full((_L,), pos, jnp.int32)], jnp.full((_L,), val), mask=lane0)


def _vec_at(vec, w):
    return jnp.max(jnp.where(_lane_iota() == w, vec, -jnp.inf))


def _sc_stage(n_chunks, B, W, V):
    n_mvecs = n_chunks // _L
    mesh = plsc.VectorSubcoreMesh(core_axis_name="c", subcore_axis_name="s")
    info = plsc.get_sparse_core_info()
    nc = info.num_cores

    def body(
        logits_hbm,
        cmax_hbm,
        lse_hbm,
        scores_hbm,
        out_s,
        out_r,
        out_t,
        mrow,
        cbuf,
        candv,
        candi,
        outv,
        outr,
        outt,
        cid_smem,
    ):
        b = lax.axis_index("s") * nc + lax.axis_index("c")

        @pl.when(b < B)
        def _work():
            _sc_batch(
                b, logits_hbm, cmax_hbm, lse_hbm, scores_hbm, out_s, out_r, out_t,
                mrow, cbuf, candv, candi, outv, outr, outt, cid_smem,
            )

    def _sc_batch(
        b, logits_hbm, cmax_hbm, lse_hbm, scores_hbm, out_s, out_r, out_t,
        mrow, cbuf, candv, candi, outv, outr, outt, cid_smem,
    ):
        pltpu.sync_copy(lse_hbm.at[b], mrow.at[pl.ds(0, _L)])
        lse_v = mrow[pl.ds(0, _L)]
        pltpu.sync_copy(scores_hbm.at[b], mrow.at[pl.ds(0, _L)])
        sc_v = mrow[pl.ds(0, _L)]

        def beam_body(w, _):
            row = b * W + w
            lse_w = _vec_at(lse_v, w)
            sc_w = _vec_at(sc_v, w)
            pltpu.sync_copy(cmax_hbm.at[row], mrow)

            def sel_body(r, __):
                _, ci = _first_argmax_scan(mrow, n_mvecs, lambda i: i * _L)
                cid_smem[r] = ci
                _mask_out(mrow, ci)
                return 0

            lax.fori_loop(0, W, sel_body, 0)

            def gat_body(k, __):
                off = cid_smem[k] * _CH
                pltpu.sync_copy(
                    logits_hbm.at[row, pl.ds(off, _CH)], cbuf.at[pl.ds(k * _CH, _CH)]
                )
                return 0

            lax.fori_loop(0, W, gat_body, 0)

            def top_body(r, __):
                def scan_body(j, carry):
                    runv, rung, runl = carry
                    v = cbuf[pl.ds(j * _L, _L)]
                    gbase = cid_smem[j // (_CH // _L)] * _CH + (j % (_CH // _L)) * _L
                    g = gbase + _lane_iota()
                    l = j * _L + _lane_iota()
                    better = (v > runv) | ((v == runv) & (g < rung))
                    return (
                        jnp.where(better, v, runv),
                        jnp.where(better, g, rung),
                        jnp.where(better, l, runl),
                    )

                init = (
                    jnp.full((_L,), -jnp.inf, jnp.float32),
                    jnp.full((_L,), jnp.int32(2**30)),
                    jnp.full((_L,), jnp.int32(2**30)),
                )
                runv, rung, runl = lax.fori_loop(0, W * _CH // _L, scan_body, init)
                mx = jnp.max(runv)
                big = jnp.int32(2**30)
                gsel = jnp.min(jnp.where(runv == mx, rung, big))
                lsel = jnp.min(jnp.where((runv == mx) & (rung == gsel), runl, big))
                _mask_out(cbuf, lsel)
                pos = w * W + r
                _scatter1(candv, pos, sc_w + mx - lse_w)
                _scatter1(candi, pos, gsel)
                return 0

            lax.fori_loop(0, W, top_body, 0)
            return 0

        lax.fori_loop(0, W, beam_body, 0)

        outv[...] = jnp.zeros((_L,), jnp.float32)
        outr[...] = jnp.zeros((_L,), jnp.int32)
        outt[...] = jnp.zeros((_L,), jnp.int32)

        def merge_body(r, __):
            mx, fi = _first_argmax_scan(candv, W * W // _L, lambda i: i * _L)
            tok16 = plsc.load_gather(candi, [jnp.full((_L,), fi, jnp.int32)])
            tok = jnp.max(tok16)
            _mask_out(candv, fi)
            _scatter1(outv, r, mx)
            _scatter1(outr, r, fi // W)
            _scatter1(outt, r, tok)
            return 0

        lax.fori_loop(0, W, merge_body, 0)
        pltpu.sync_copy(outv, out_s.at[b])
        pltpu.sync_copy(outr, out_r.at[b])
        pltpu.sync_copy(outt, out_t.at[b])

    return mesh, body


def kernel(scores, logits, beam_width):
    del beam_width
    B, W, V = logits.shape
    n_chunks = ((-(-V // _CH) + _L - 1) // _L) * _L
    lse3, cmax3 = pl.pallas_call(
        functools.partial(_dense_kernel, n_chunks=n_chunks),
        grid=(B,),
        in_specs=[pl.BlockSpec((1, W, V), lambda b: (b, 0, 0))],
        out_specs=(
            pl.BlockSpec((1, W, 1), lambda b: (b, 0, 0)),
            pl.BlockSpec((1, W, n_chunks), lambda b: (b, 0, 0)),
        ),
        out_shape=(
            jax.ShapeDtypeStruct((B, W, 1), jnp.float32),
            jax.ShapeDtypeStruct((B, W, n_chunks), jnp.float32),
        ),
    )(logits)

    logits2d = logits.reshape(B * W, V)
    cmax2d = cmax3.reshape(B * W, n_chunks)
    lse_p = jnp.concatenate(
        [lse3.reshape(B, W), jnp.zeros((B, _L - W), jnp.float32)], axis=1
    )
    sc_p = jnp.concatenate([scores, jnp.zeros((B, _L - W), jnp.float32)], axis=1)

    mesh, body = _sc_stage(n_chunks, B, W, V)
    out_s, out_r, out_t = pl.kernel(
        body,
        mesh=mesh,
        compiler_params=pltpu.CompilerParams(
            needs_layout_passes=False, use_tc_tiling_on_sc=False
        ),
        out_type=(
            jax.ShapeDtypeStruct((B, _L), jnp.float32),
            jax.ShapeDtypeStruct((B, _L), jnp.int32),
            jax.ShapeDtypeStruct((B, _L), jnp.int32),
        ),
        scratch_types=[
            pltpu.VMEM((n_chunks,), jnp.float32),
            pltpu.VMEM((W * _CH,), jnp.float32),
            pltpu.VMEM((W * W,), jnp.float32),
            pltpu.VMEM((W * W,), jnp.int32),
            pltpu.VMEM((_L,), jnp.float32),
            pltpu.VMEM((_L,), jnp.int32),
            pltpu.VMEM((_L,), jnp.int32),
            pltpu.SMEM((W,), jnp.int32),
        ],
    )(logits2d, cmax2d, lse_p, sc_p)
    return out_s[:, :W], out_r[:, :W], out_t[:, :W]


# speedup vs baseline: 77.2572x; 1.0002x over previous
import functools

import jax
import jax.numpy as jnp
from jax import lax
from jax.experimental import pallas as pl
from jax.experimental.pallas import tpu as pltpu
from jax.experimental.pallas import tpu_sc as plsc

_CH = 128
_L = 16


def _dense_kernel(logits_ref, lse_ref, cmax_ref, *, n_chunks):
    x = logits_ref[0]
    W, V = x.shape
    neg = jnp.float32(-jnp.inf)
    nfull = V // _CH
    cm_main = jnp.max(x[:, : nfull * _CH].reshape(W, nfull, _CH), axis=-1)
    parts = [cm_main]
    if nfull * _CH < V:
        parts.append(jnp.max(x[:, nfull * _CH :], axis=-1, keepdims=True))
    if len(parts) == 1 or n_chunks > nfull + 1:
        parts.append(jnp.full((W, n_chunks - len(parts[1:]) - nfull), neg, jnp.float32))
    cmax = jnp.concatenate(parts, axis=1)
    cmax_ref[0] = cmax
    m = jnp.max(cmax, axis=-1, keepdims=True)
    s = jnp.sum(jnp.exp(x - m), axis=-1, keepdims=True)
    lse_ref[0] = m + jnp.log(s)


def _lane_iota():
    return lax.iota(jnp.int32, _L)


def _first_argmax_scan(ref, n_vecs, idx_base_fn):
    def body(i, carry):
        runv, runi = carry
        v = ref[pl.ds(i * _L, _L)]
        idx = idx_base_fn(i) + _lane_iota()
        better = (v > runv) | ((v == runv) & (idx < runi))
        return jnp.where(better, v, runv), jnp.where(better, idx, runi)

    init = (jnp.full((_L,), -jnp.inf, jnp.float32), jnp.full((_L,), jnp.int32(2**30)))
    runv, runi = lax.fori_loop(0, n_vecs, body, init)
    mx = jnp.max(runv)
    mi = jnp.min(jnp.where(runv == mx, runi, jnp.int32(2**30)))
    return mx, mi


def _mask_out(ref, pos):
    lane0 = _lane_iota() == 0
    plsc.store_scatter(
        ref,
        [jnp.full((_L,), pos, jnp.int32)],
        jnp.full((_L,), -jnp.inf, jnp.float32),
        mask=lane0,
    )


def _scatter1(ref, pos, val):
    lane0 = _lane_iota() == 0
    plsc.store_scatter(ref, [jnp.full((_L,), pos, jnp.int32)], jnp.full((_L,), val), mask=lane0)


def _vec_at(vec, w):
    return jnp.max(jnp.where(_lane_iota() == w, vec, -jnp.inf))


def _sc_stage(n_chunks, B, W, V):
    n_mvecs = n_chunks // _L
    mesh = plsc.VectorSubcoreMesh(core_axis_name="c", subcore_axis_name="s")
    info = plsc.get_sparse_core_info()
    nc = info.num_cores

    def body(
        logits_hbm,
        cmax_hbm,
        lse_hbm,
        scores_hbm,
        out_s,
        out_r,
        out_t,
        mrow,
        cbuf,
        candv,
        candi,
        outv,
        outr,
        outt,
        cid_smem,
    ):
        b = lax.axis_index("s") * nc + lax.axis_index("c")

        @pl.when(b < B)
        def _work():
            _sc_batch(
                b, logits_hbm, cmax_hbm, lse_hbm, scores_hbm, out_s, out_r, out_t,
                mrow, cbuf, candv, candi, outv, outr, outt, cid_smem,
            )

    def _sc_batch(
        b, logits_hbm, cmax_hbm, lse_hbm, scores_hbm, out_s, out_r, out_t,
        mrow, cbuf, candv, candi, outv, outr, outt, cid_smem,
    ):
        pltpu.sync_copy(lse_hbm.at[b], mrow.at[pl.ds(0, _L)])
        lse_v = mrow[pl.ds(0, _L)]
        pltpu.sync_copy(scores_hbm.at[b], mrow.at[pl.ds(0, _L)])
        sc_v = mrow[pl.ds(0, _L)]

        def beam_body(w, _):
            row = b * W + w
            lse_w = _vec_at(lse_v, w)
            sc_w = _vec_at(sc_v, w)
            pltpu.sync_copy(cmax_hbm.at[row], mrow)

            def sel_body(r, __):
                _, ci = _first_argmax_scan(mrow, n_mvecs, lambda i: i * _L)
                cid_smem[r] = ci
                _mask_out(mrow, ci)
                return 0

            lax.fori_loop(0, W, sel_body, 0)

            def gat_body(k, __):
                off = cid_smem[k] * _CH
                pltpu.sync_copy(
                    logits_hbm.at[row, pl.ds(off, _CH)], cbuf.at[pl.ds(k * _CH, _CH)]
                )
                return 0

            lax.fori_loop(0, W, gat_body, 0)

            def top_body(r, __):
                def scan_body(j, carry):
                    runv, rung, runl = carry
                    v = cbuf[pl.ds(j * _L, _L)]
                    gbase = cid_smem[j // (_CH // _L)] * _CH + (j % (_CH // _L)) * _L
                    g = gbase + _lane_iota()
                    l = j * _L + _lane_iota()
                    better = (v > runv) | ((v == runv) & (g < rung))
                    return (
                        jnp.where(better, v, runv),
                        jnp.where(better, g, rung),
                        jnp.where(better, l, runl),
                    )

                init = (
                    jnp.full((_L,), -jnp.inf, jnp.float32),
                    jnp.full((_L,), jnp.int32(2**30)),
                    jnp.full((_L,), jnp.int32(2**30)),
                )
                runv, rung, runl = lax.fori_loop(0, W * _CH // _L, scan_body, init)
                mx = jnp.max(runv)
                big = jnp.int32(2**30)
                gsel = jnp.min(jnp.where(runv == mx, rung, big))
                lsel = jnp.min(jnp.where((runv == mx) & (rung == gsel), runl, big))
                _mask_out(cbuf, lsel)
                pos = w * W + r
                _scatter1(candv, pos, sc_w + mx - lse_w)
                _scatter1(candi, pos, gsel)
                return 0

            lax.fori_loop(0, W, top_body, 0)
            return 0

        lax.fori_loop(0, W, beam_body, 0)

        outv[...] = jnp.zeros((_L,), jnp.float32)
        outr[...] = jnp.zeros((_L,), jnp.int32)
        outt[...] = jnp.zeros((_L,), jnp.int32)

        def merge_body(r, __):
            mx, fi = _first_argmax_scan(candv, W * W // _L, lambda i: i * _L)
            tok16 = plsc.load_gather(candi, [jnp.full((_L,), fi, jnp.int32)])
            tok = jnp.max(tok16)
            _mask_out(candv, fi)
            _scatter1(outv, r, mx)
            _scatter1(outr, r, fi // W)
            _scatter1(outt, r, tok)
            return 0

        lax.fori_loop(0, W, merge_body, 0)
        pltpu.sync_copy(outv, out_s.at[b])
        pltpu.sync_copy(outr, out_r.at[b])
        pltpu.sync_copy(outt, out_t.at[b])

    return mesh, body


def kernel(scores, logits, beam_width):
    del beam_width
    B, W, V = logits.shape
    n_chunks = ((-(-V // _CH) + _L - 1) // _L) * _L
    lse3, cmax3 = pl.pallas_call(
        functools.partial(_dense_kernel, n_chunks=n_chunks),
        grid=(B,),
        in_specs=[pl.BlockSpec((1, W, V), lambda b: (b, 0, 0))],
        out_specs=(
            pl.BlockSpec((1, W, 1), lambda b: (b, 0, 0)),
            pl.BlockSpec((1, W, n_chunks), lambda b: (b, 0, 0)),
        ),
        out_shape=(
            jax.ShapeDtypeStruct((B, W, 1), jnp.float32),
            jax.ShapeDtypeStruct((B, W, n_chunks), jnp.float32),
        ),
    )(logits)

    logits2d = logits.reshape(B * W, V)
    cmax2d = cmax3.reshape(B * W, n_chunks)
    lse_p = jnp.concatenate(
        [lse3.reshape(B, W), jnp.zeros((B, _L - W), jnp.float32)], axis=1
    )
    sc_p = jnp.concatenate([scores, jnp.zeros((B, _L - W), jnp.float32)], axis=1)

    mesh, body = _sc_stage(n_chunks, B, W, V)
    out_s, out_r, out_t = pl.kernel(
        body,
        mesh=mesh,
        compiler_params=pltpu.CompilerParams(
            needs_layout_passes=False, use_tc_tiling_on_sc=False
        ),
        out_type=(
            jax.ShapeDtypeStruct((B, _L), jnp.float32),
            jax.ShapeDtypeStruct((B, _L), jnp.int32),
            jax.ShapeDtypeStruct((B, _L), jnp.int32),
        ),
        scratch_types=[
            pltpu.VMEM((n_chunks,), jnp.float32),
            pltpu.VMEM((W * _CH,), jnp.float32),
            pltpu.VMEM((W * W,), jnp.float32),
            pltpu.VMEM((W * W,), jnp.int32),
            pltpu.VMEM((_L,), jnp.float32),
            pltpu.VMEM((_L,), jnp.int32),
            pltpu.VMEM((_L,), jnp.int32),
            pltpu.SMEM((W,), jnp.int32),
        ],
    )(logits2d, cmax2d, lse_p, sc_p)
    return out_s[:, :W], out_r[:, :W], out_t[:, :W]


# speedup vs baseline: 83.9101x; 1.0861x over previous
import functools

import jax
import jax.numpy as jnp
from jax import lax
from jax.experimental import pallas as pl
from jax.experimental.pallas import tpu as pltpu
from jax.experimental.pallas import tpu_sc as plsc

_CH = 128
_L = 16


def _dense_kernel(logits_ref, lse_ref, cmax_ref, *, n_chunks):
    x = logits_ref[0]
    W, V = x.shape
    neg = jnp.float32(-jnp.inf)
    nfull = V // _CH
    cm_main = jnp.max(x[:, : nfull * _CH].reshape(W, nfull, _CH), axis=-1)
    parts = [cm_main]
    if nfull * _CH < V:
        parts.append(jnp.max(x[:, nfull * _CH :], axis=-1, keepdims=True))
    if len(parts) == 1 or n_chunks > nfull + 1:
        parts.append(jnp.full((W, n_chunks - len(parts[1:]) - nfull), neg, jnp.float32))
    cmax = jnp.concatenate(parts, axis=1)
    cmax_ref[0] = cmax
    m = jnp.max(cmax, axis=-1, keepdims=True)
    s = jnp.sum(jnp.exp(x - m), axis=-1, keepdims=True)
    lse_ref[0] = m + jnp.log(s)


def _lane_iota():
    return lax.iota(jnp.int32, _L)


def _first_argmax_scan(ref, n_vecs, idx_base_fn):
    def body(i, carry):
        runv, runi = carry
        v = ref[pl.ds(i * _L, _L)]
        idx = idx_base_fn(i) + _lane_iota()
        better = (v > runv) | ((v == runv) & (idx < runi))
        return jnp.where(better, v, runv), jnp.where(better, idx, runi)

    init = (jnp.full((_L,), -jnp.inf, jnp.float32), jnp.full((_L,), jnp.int32(2**30)))
    runv, runi = lax.fori_loop(0, n_vecs, body, init)
    mx = jnp.max(runv)
    mi = jnp.min(jnp.where(runv == mx, runi, jnp.int32(2**30)))
    return mx, mi


def _mask_out(ref, pos):
    lane0 = _lane_iota() == 0
    plsc.store_scatter(
        ref,
        [jnp.full((_L,), pos, jnp.int32)],
        jnp.full((_L,), -jnp.inf, jnp.float32),
        mask=lane0,
    )


def _scatter1(ref, pos, val):
    lane0 = _lane_iota() == 0
    plsc.store_scatter(ref, [jnp.full((_L,), pos, jnp.int32)], jnp.full((_L,), val), mask=lane0)


def _vec_at(vec, w):
    return jnp.max(jnp.where(_lane_iota() == w, vec, -jnp.inf))


def _sc_stage(n_chunks, B, W, V):
    n_mvecs = n_chunks // _L
    mesh = plsc.VectorSubcoreMesh(core_axis_name="c", subcore_axis_name="s")
    info = plsc.get_sparse_core_info()
    nc = info.num_cores

    def body(
        logits_hbm,
        cmax_hbm,
        lse_hbm,
        scores_hbm,
        out_s,
        out_r,
        out_t,
        mrow,
        cbuf,
        candv,
        candi,
        outv,
        outr,
        outt,
        cid_smem,
        dma_sem,
    ):
        b = lax.axis_index("s") * nc + lax.axis_index("c")

        @pl.when(b < B)
        def _work():
            _sc_batch(
                b, logits_hbm, cmax_hbm, lse_hbm, scores_hbm, out_s, out_r, out_t,
                mrow, cbuf, candv, candi, outv, outr, outt, cid_smem, dma_sem,
            )

    def _sc_batch(
        b, logits_hbm, cmax_hbm, lse_hbm, scores_hbm, out_s, out_r, out_t,
        mrow, cbuf, candv, candi, outv, outr, outt, cid_smem, dma_sem,
    ):
        pltpu.sync_copy(lse_hbm.at[b], mrow.at[pl.ds(0, _L)])
        lse_v = mrow[pl.ds(0, _L)]
        pltpu.sync_copy(scores_hbm.at[b], mrow.at[pl.ds(0, _L)])
        sc_v = mrow[pl.ds(0, _L)]

        def beam_body(w, _):
            row = b * W + w
            lse_w = _vec_at(lse_v, w)
            sc_w = _vec_at(sc_v, w)
            pltpu.sync_copy(cmax_hbm.at[row], mrow)

            def sel_body(r, __):
                _, ci = _first_argmax_scan(mrow, n_mvecs, lambda i: i * _L)
                cid_smem[r] = ci
                _mask_out(mrow, ci)
                return 0

            lax.fori_loop(0, W, sel_body, 0)

            copies = []
            for k in range(W):
                off = cid_smem[k] * _CH
                copies.append(
                    pltpu.async_copy(
                        logits_hbm.at[row, pl.ds(off, _CH)],
                        cbuf.at[pl.ds(k * _CH, _CH)],
                        dma_sem,
                    )
                )
            for c in copies:
                c.wait()

            def top_body(r, __):
                def scan_body(j, carry):
                    runv, rung, runl = carry
                    v = cbuf[pl.ds(j * _L, _L)]
                    gbase = cid_smem[j // (_CH // _L)] * _CH + (j % (_CH // _L)) * _L
                    g = gbase + _lane_iota()
                    l = j * _L + _lane_iota()
                    better = (v > runv) | ((v == runv) & (g < rung))
                    return (
                        jnp.where(better, v, runv),
                        jnp.where(better, g, rung),
                        jnp.where(better, l, runl),
                    )

                init = (
                    jnp.full((_L,), -jnp.inf, jnp.float32),
                    jnp.full((_L,), jnp.int32(2**30)),
                    jnp.full((_L,), jnp.int32(2**30)),
                )
                runv, rung, runl = lax.fori_loop(0, W * _CH // _L, scan_body, init)
                mx = jnp.max(runv)
                big = jnp.int32(2**30)
                gsel = jnp.min(jnp.where(runv == mx, rung, big))
                lsel = jnp.min(jnp.where((runv == mx) & (rung == gsel), runl, big))
                _mask_out(cbuf, lsel)
                pos = w * W + r
                _scatter1(candv, pos, sc_w + mx - lse_w)
                _scatter1(candi, pos, gsel)
                return 0

            lax.fori_loop(0, W, top_body, 0)
            return 0

        lax.fori_loop(0, W, beam_body, 0)

        outv[...] = jnp.zeros((_L,), jnp.float32)
        outr[...] = jnp.zeros((_L,), jnp.int32)
        outt[...] = jnp.zeros((_L,), jnp.int32)

        def merge_body(r, __):
            mx, fi = _first_argmax_scan(candv, W * W // _L, lambda i: i * _L)
            tok16 = plsc.load_gather(candi, [jnp.full((_L,), fi, jnp.int32)])
            tok = jnp.max(tok16)
            _mask_out(candv, fi)
            _scatter1(outv, r, mx)
            _scatter1(outr, r, fi // W)
            _scatter1(outt, r, tok)
            return 0

        lax.fori_loop(0, W, merge_body, 0)
        pltpu.sync_copy(outv, out_s.at[b])
        pltpu.sync_copy(outr, out_r.at[b])
        pltpu.sync_copy(outt, out_t.at[b])

    return mesh, body


def kernel(scores, logits, beam_width):
    del beam_width
    B, W, V = logits.shape
    n_chunks = ((-(-V // _CH) + _L - 1) // _L) * _L
    lse3, cmax3 = pl.pallas_call(
        functools.partial(_dense_kernel, n_chunks=n_chunks),
        grid=(B,),
        in_specs=[pl.BlockSpec((1, W, V), lambda b: (b, 0, 0))],
        out_specs=(
            pl.BlockSpec((1, W, 1), lambda b: (b, 0, 0)),
            pl.BlockSpec((1, W, n_chunks), lambda b: (b, 0, 0)),
        ),
        out_shape=(
            jax.ShapeDtypeStruct((B, W, 1), jnp.float32),
            jax.ShapeDtypeStruct((B, W, n_chunks), jnp.float32),
        ),
    )(logits)

    logits2d = logits.reshape(B * W, V)
    cmax2d = cmax3.reshape(B * W, n_chunks)
    lse_p = jnp.concatenate(
        [lse3.reshape(B, W), jnp.zeros((B, _L - W), jnp.float32)], axis=1
    )
    sc_p = jnp.concatenate([scores, jnp.zeros((B, _L - W), jnp.float32)], axis=1)

    mesh, body = _sc_stage(n_chunks, B, W, V)
    out_s, out_r, out_t = pl.kernel(
        body,
        mesh=mesh,
        compiler_params=pltpu.CompilerParams(
            needs_layout_passes=False, use_tc_tiling_on_sc=False
        ),
        out_type=(
            jax.ShapeDtypeStruct((B, _L), jnp.float32),
            jax.ShapeDtypeStruct((B, _L), jnp.int32),
            jax.ShapeDtypeStruct((B, _L), jnp.int32),
        ),
        scratch_types=[
            pltpu.VMEM((n_chunks,), jnp.float32),
            pltpu.VMEM((W * _CH,), jnp.float32),
            pltpu.VMEM((W * W,), jnp.float32),
            pltpu.VMEM((W * W,), jnp.int32),
            pltpu.VMEM((_L,), jnp.float32),
            pltpu.VMEM((_L,), jnp.int32),
            pltpu.VMEM((_L,), jnp.int32),
            pltpu.SMEM((W,), jnp.int32),
            pltpu.SemaphoreType.DMA,
        ],
    )(logits2d, cmax2d, lse_p, sc_p)
    return out_s[:, :W], out_r[:, :W], out_t[:, :W]


# speedup vs baseline: 84.8004x; 1.0106x over previous
import functools

import jax
import jax.numpy as jnp
from jax import lax
from jax.experimental import pallas as pl
from jax.experimental.pallas import tpu as pltpu
from jax.experimental.pallas import tpu_sc as plsc

_CH = 128
_L = 16


def _dense_kernel(logits_ref, lse_ref, cmax_ref, *, n_chunks):
    x = logits_ref[0]
    W, V = x.shape
    neg = jnp.float32(-jnp.inf)
    nfull = V // _CH
    cm_main = jnp.max(x[:, : nfull * _CH].reshape(W, nfull, _CH), axis=-1)
    parts = [cm_main]
    if nfull * _CH < V:
        parts.append(jnp.max(x[:, nfull * _CH :], axis=-1, keepdims=True))
    if len(parts) == 1 or n_chunks > nfull + 1:
        parts.append(jnp.full((W, n_chunks - len(parts[1:]) - nfull), neg, jnp.float32))
    cmax = jnp.concatenate(parts, axis=1)
    cmax_ref[0] = cmax
    m = jnp.max(cmax, axis=-1, keepdims=True)
    s = jnp.sum(jnp.exp(x - m), axis=-1, keepdims=True)
    lse_ref[0] = m + jnp.log(s)


def _lane_iota():
    return lax.iota(jnp.int32, _L)


def _first_argmax_scan(ref, n_vecs, idx_base_fn):
    def body(i, carry):
        runv, runi = carry
        v = ref[pl.ds(i * _L, _L)]
        idx = idx_base_fn(i) + _lane_iota()
        better = (v > runv) | ((v == runv) & (idx < runi))
        return jnp.where(better, v, runv), jnp.where(better, idx, runi)

    init = (jnp.full((_L,), -jnp.inf, jnp.float32), jnp.full((_L,), jnp.int32(2**30)))
    runv, runi = lax.fori_loop(0, n_vecs, body, init)
    mx = jnp.max(runv)
    mi = jnp.min(jnp.where(runv == mx, runi, jnp.int32(2**30)))
    return mx, mi


def _mask_out(ref, pos):
    lane0 = _lane_iota() == 0
    plsc.store_scatter(
        ref,
        [jnp.full((_L,), pos, jnp.int32)],
        jnp.full((_L,), -jnp.inf, jnp.float32),
        mask=lane0,
    )


def _scatter1(ref, pos, val):
    lane0 = _lane_iota() == 0
    plsc.store_scatter(ref, [jnp.full((_L,), pos, jnp.int32)], jnp.full((_L,), val), mask=lane0)


def _vec_at(vec, w):
    return jnp.max(jnp.where(_lane_iota() == w, vec, -jnp.inf))


def _sc_stage(n_chunks, B, W, V):
    n_mvecs = n_chunks // _L
    mesh = plsc.VectorSubcoreMesh(core_axis_name="c", subcore_axis_name="s")
    info = plsc.get_sparse_core_info()
    nc = info.num_cores

    def body(
        logits_hbm,
        cmax_hbm,
        lse_hbm,
        scores_hbm,
        out_s,
        out_r,
        out_t,
        mrow,
        cbuf,
        candv,
        candi,
        outv,
        outr,
        outt,
        cid_smem,
        dma_sem,
    ):
        b = lax.axis_index("s") * nc + lax.axis_index("c")

        @pl.when(b < B)
        def _work():
            _sc_batch(
                b, logits_hbm, cmax_hbm, lse_hbm, scores_hbm, out_s, out_r, out_t,
                mrow, cbuf, candv, candi, outv, outr, outt, cid_smem, dma_sem,
            )

    def _sc_batch(
        b, logits_hbm, cmax_hbm, lse_hbm, scores_hbm, out_s, out_r, out_t,
        mrow, cbuf, candv, candi, outv, outr, outt, cid_smem, dma_sem,
    ):
        pltpu.sync_copy(lse_hbm.at[b], mrow.at[pl.ds(0, _L)])
        lse_v = mrow[pl.ds(0, _L)]
        pltpu.sync_copy(scores_hbm.at[b], mrow.at[pl.ds(0, _L)])
        sc_v = mrow[pl.ds(0, _L)]

        def beam_body(w, _):
            lse_w = _vec_at(lse_v, w)
            sc_w = _vec_at(sc_v, w)
            pltpu.sync_copy(cmax_hbm.at[b, w], mrow)

            def sel_body(r, __):
                _, ci = _first_argmax_scan(mrow, n_mvecs, lambda i: i * _L)
                cid_smem[r] = ci
                _mask_out(mrow, ci)
                return 0

            lax.fori_loop(0, W, sel_body, 0)

            copies = []
            for k in range(W):
                off = cid_smem[k] * _CH
                copies.append(
                    pltpu.async_copy(
                        logits_hbm.at[b, w, pl.ds(off, _CH)],
                        cbuf.at[pl.ds(k * _CH, _CH)],
                        dma_sem,
                    )
                )
            for c in copies:
                c.wait()

            def top_body(r, __):
                def scan_body(j, carry):
                    runv, rung, runl = carry
                    v = cbuf[pl.ds(j * _L, _L)]
                    gbase = cid_smem[j // (_CH // _L)] * _CH + (j % (_CH // _L)) * _L
                    g = gbase + _lane_iota()
                    l = j * _L + _lane_iota()
                    better = (v > runv) | ((v == runv) & (g < rung))
                    return (
                        jnp.where(better, v, runv),
                        jnp.where(better, g, rung),
                        jnp.where(better, l, runl),
                    )

                init = (
                    jnp.full((_L,), -jnp.inf, jnp.float32),
                    jnp.full((_L,), jnp.int32(2**30)),
                    jnp.full((_L,), jnp.int32(2**30)),
                )
                runv, rung, runl = lax.fori_loop(0, W * _CH // _L, scan_body, init)
                mx = jnp.max(runv)
                big = jnp.int32(2**30)
                gsel = jnp.min(jnp.where(runv == mx, rung, big))
                lsel = jnp.min(jnp.where((runv == mx) & (rung == gsel), runl, big))
                _mask_out(cbuf, lsel)
                pos = w * W + r
                _scatter1(candv, pos, sc_w + mx - lse_w)
                _scatter1(candi, pos, gsel)
                return 0

            lax.fori_loop(0, W, top_body, 0)
            return 0

        lax.fori_loop(0, W, beam_body, 0)

        outv[...] = jnp.zeros((_L,), jnp.float32)
        outr[...] = jnp.zeros((_L,), jnp.int32)
        outt[...] = jnp.zeros((_L,), jnp.int32)

        def merge_body(r, __):
            mx, fi = _first_argmax_scan(candv, W * W // _L, lambda i: i * _L)
            tok16 = plsc.load_gather(candi, [jnp.full((_L,), fi, jnp.int32)])
            tok = jnp.max(tok16)
            _mask_out(candv, fi)
            _scatter1(outv, r, mx)
            _scatter1(outr, r, fi // W)
            _scatter1(outt, r, tok)
            return 0

        lax.fori_loop(0, W, merge_body, 0)
        pltpu.sync_copy(outv, out_s.at[b])
        pltpu.sync_copy(outr, out_r.at[b])
        pltpu.sync_copy(outt, out_t.at[b])

    return mesh, body


def kernel(scores, logits, beam_width):
    del beam_width
    B, W, V = logits.shape
    n_chunks = ((-(-V // _CH) + _L - 1) // _L) * _L
    lse3, cmax3 = pl.pallas_call(
        functools.partial(_dense_kernel, n_chunks=n_chunks),
        grid=(B,),
        in_specs=[pl.BlockSpec((1, W, V), lambda b: (b, 0, 0))],
        out_specs=(
            pl.BlockSpec((1, W, 1), lambda b: (b, 0, 0)),
            pl.BlockSpec((1, W, n_chunks), lambda b: (b, 0, 0)),
        ),
        out_shape=(
            jax.ShapeDtypeStruct((B, W, 1), jnp.float32),
            jax.ShapeDtypeStruct((B, W, n_chunks), jnp.float32),
        ),
    )(logits)

    lse_p = jnp.concatenate(
        [lse3.reshape(B, W), jnp.zeros((B, _L - W), jnp.float32)], axis=1
    )
    sc_p = jnp.concatenate([scores, jnp.zeros((B, _L - W), jnp.float32)], axis=1)

    mesh, body = _sc_stage(n_chunks, B, W, V)
    out_s, out_r, out_t = pl.kernel(
        body,
        mesh=mesh,
        compiler_params=pltpu.CompilerParams(
            needs_layout_passes=False, use_tc_tiling_on_sc=False
        ),
        out_type=(
            jax.ShapeDtypeStruct((B, _L), jnp.float32),
            jax.ShapeDtypeStruct((B, _L), jnp.int32),
            jax.ShapeDtypeStruct((B, _L), jnp.int32),
        ),
        scratch_types=[
            pltpu.VMEM((n_chunks,), jnp.float32),
            pltpu.VMEM((W * _CH,), jnp.float32),
            pltpu.VMEM((W * W,), jnp.float32),
            pltpu.VMEM((W * W,), jnp.int32),
            pltpu.VMEM((_L,), jnp.float32),
            pltpu.VMEM((_L,), jnp.int32),
            pltpu.VMEM((_L,), jnp.int32),
            pltpu.SMEM((W,), jnp.int32),
            pltpu.SemaphoreType.DMA,
        ],
    )(logits, cmax3, lse_p, sc_p)
    return out_s[:, :W], out_r[:, :W], out_t[:, :W]
